# baseline (speedup 1.0000x reference)
import jax
import jax.numpy as jnp
from jax.experimental import pallas as pl
from jax.experimental.pallas import tpu as pltpu

_BOUND = 512.0
_NCH = 8
_BUF = 4


def _fsq_body(z_hbm, out_hbm, loss_ref, in_buf, out_buf, acc_ref,
              in_sems, out_sems):
    ch = z_hbm.shape[0] // _NCH

    def in_copy(c, slot):
        return pltpu.make_async_copy(
            z_hbm.at[pl.ds(c * ch, ch)], in_buf.at[slot], in_sems.at[slot])

    def out_copy(c, slot):
        return pltpu.make_async_copy(
            out_buf.at[slot], out_hbm.at[pl.ds(c * ch, ch)], out_sems.at[slot])

    for c in range(_BUF):
        in_copy(c, c).start()
    for c in range(_NCH):
        slot = c % _BUF
        in_copy(c, slot).wait()
        z = in_buf[slot]
        qz = jnp.round(_BOUND * jnp.tanh(z))
        delta = qz - z
        if c >= _BUF:
            out_copy(c - _BUF, slot).wait()
        out_buf[slot] = qz
        out_copy(c, slot).start()
        part = jnp.sum(delta * delta, axis=0)
        if c == 0:
            acc_ref[...] = part
        else:
            acc_ref[...] += part
        if c + _BUF < _NCH:
            in_copy(c + _BUF, slot).start()
    for c in range(_NCH - _BUF, _NCH):
        out_copy(c, c % _BUF).wait()
    loss_ref[0, 0] = jnp.sum(acc_ref[...])


def kernel(z, codebook):
    del codebook
    b, s, d = z.shape
    n = z.size
    zt = jnp.transpose(z, (0, 2, 1))
    ch = b // _NCH
    out_t, loss = pl.pallas_call(
        _fsq_body,
        compiler_params=pltpu.CompilerParams(
            vmem_limit_bytes=57 * 1024 * 1024,
        ),
        in_specs=[pl.BlockSpec(memory_space=pl.ANY)],
        out_specs=[
            pl.BlockSpec(memory_space=pl.ANY),
            pl.BlockSpec(memory_space=pltpu.SMEM),
        ],
        out_shape=[
            jax.ShapeDtypeStruct((b, d, s), z.dtype),
            jax.ShapeDtypeStruct((1, 1), jnp.float32),
        ],
        scratch_shapes=[
            pltpu.VMEM((_BUF, ch, d, s), jnp.float32),
            pltpu.VMEM((_BUF, ch, d, s), jnp.float32),
            pltpu.VMEM((d, s), jnp.float32),
            pltpu.SemaphoreType.DMA((_BUF,)),
            pltpu.SemaphoreType.DMA((_BUF,)),
        ],
    )(zt)
    total_loss = loss[0, 0] * (2.0 / n)
    return (jnp.transpose(out_t, (0, 2, 1)), total_loss)


# speedup vs baseline: 1.1471x; 1.1471x over previous
import jax
import jax.numpy as jnp
from jax.experimental import pallas as pl
from jax.experimental.pallas import tpu as pltpu

_BOUND = 512.0
_NCH = 4
_BUF = 4


def _fsq_body(z_hbm, out_hbm, loss_ref, in_buf, out_buf, acc_ref,
              in_sems, out_sems):
    ch = z_hbm.shape[0] // _NCH

    def in_copy(c, slot):
        return pltpu.make_async_copy(
            z_hbm.at[pl.ds(c * ch, ch)], in_buf.at[slot], in_sems.at[slot])

    def out_copy(c, slot):
        return pltpu.make_async_copy(
            out_buf.at[slot], out_hbm.at[pl.ds(c * ch, ch)], out_sems.at[slot])

    for c in range(_BUF):
        in_copy(c, c).start()
    for c in range(_NCH):
        slot = c % _BUF
        in_copy(c, slot).wait()
        z = in_buf[slot]
        qz = jnp.round(_BOUND * jnp.tanh(z))
        delta = qz - z
        if c >= _BUF:
            out_copy(c - _BUF, slot).wait()
        out_buf[slot] = qz
        out_copy(c, slot).start()
        part = jnp.sum(delta * delta, axis=0)
        if c == 0:
            acc_ref[...] = part
        else:
            acc_ref[...] += part
        if c + _BUF < _NCH:
            in_copy(c + _BUF, slot).start()
    for c in range(_NCH - _BUF, _NCH):
        out_copy(c, c % _BUF).wait()
    loss_ref[0, 0] = jnp.sum(acc_ref[...])


def kernel(z, codebook):
    del codebook
    b, s, d = z.shape
    n = z.size
    zt = jnp.transpose(z, (0, 2, 1))
    ch = b // _NCH
    out_t, loss = pl.pallas_call(
        _fsq_body,
        compiler_params=pltpu.CompilerParams(
            vmem_limit_bytes=57 * 1024 * 1024,
        ),
        in_specs=[pl.BlockSpec(memory_space=pl.ANY)],
        out_specs=[
            pl.BlockSpec(memory_space=pl.ANY),
            pl.BlockSpec(memory_space=pltpu.SMEM),
        ],
        out_shape=[
            jax.ShapeDtypeStruct((b, d, s), z.dtype),
            jax.ShapeDtypeStruct((1, 1), jnp.float32),
        ],
        scratch_shapes=[
            pltpu.VMEM((_BUF, ch, d, s), jnp.float32),
            pltpu.VMEM((_BUF, ch, d, s), jnp.float32),
            pltpu.VMEM((d, s), jnp.float32),
            pltpu.SemaphoreType.DMA((_BUF,)),
            pltpu.SemaphoreType.DMA((_BUF,)),
        ],
    )(zt)
    total_loss = loss[0, 0] * (2.0 / n)
    return (jnp.transpose(out_t, (0, 2, 1)), total_loss)
